# baseline (speedup 1.0000x reference)
import functools

import jax
import jax.numpy as jnp
from jax.experimental import pallas as pl
from jax.experimental.pallas import tpu as pltpu
from jax.experimental.pallas import tpu_sc as plsc

_B = 4
_N = 65536
_C = 128
_M = 16
_LANES = 128
_ROWS = _N // _LANES


def _fps_body(loc_ref, idx_ref, pts_ref):
    xs = loc_ref[0, 0]
    ys = loc_ref[0, 1]
    zs = loc_ref[0, 2]
    shape = (_ROWS, _LANES)
    row = jax.lax.broadcasted_iota(jnp.int32, shape, 0)
    col = jax.lax.broadcasted_iota(jnp.int32, shape, 1)
    lin = row * _LANES + col
    big = jnp.int32(jnp.iinfo(jnp.int32).max)

    def extract(mask, v):
        return jnp.sum(jnp.where(mask, v, 0.0))

    def put_pt(i, lx, ly, lz):
        pts_ref[:, :, 0:1, i:i + 1] = lx.reshape(1, 1, 1, 1)
        pts_ref[:, :, 1:2, i:i + 1] = ly.reshape(1, 1, 1, 1)
        pts_ref[:, :, 2:3, i:i + 1] = lz.reshape(1, 1, 1, 1)

    dists = jnp.full(shape, 1e10, dtype=jnp.float32)
    cur = jnp.zeros((), dtype=jnp.int32)
    idx_ref[:, :, 0:1] = jnp.zeros((1, 1, 1), jnp.int32)

    for i in range(1, _M):
        mask = lin == cur
        lx = extract(mask, xs)
        ly = extract(mask, ys)
        lz = extract(mask, zs)
        put_pt(i - 1, lx, ly, lz)
        dx = xs - lx
        dy = ys - ly
        dz = zs - lz
        d = (dx * dx + dy * dy) + dz * dz
        dists = jnp.minimum(dists, d)
        m = jnp.max(dists)
        nxt = jnp.min(jnp.where(dists == m, lin, big)).astype(jnp.int32)
        idx_ref[:, :, i:i + 1] = nxt.reshape(1, 1, 1)
        cur = nxt

    mask = lin == cur
    put_pt(_M - 1, extract(mask, xs), extract(mask, ys), extract(mask, zs))


_SC_NC = 2
_SC_NS = 16
_SC_L = 16
_GRAN = 16
_NROWS = _N // _GRAN
_PAIRS = _B * _M
_NW = _SC_NC * _SC_NS
_PPW = _PAIRS // _NW


def _sc_gather_features(features, anchor_idx):
    table = features.reshape(_B * _C, _N)
    idx_flat = anchor_idx.reshape(_PAIRS)
    mesh = plsc.VectorSubcoreMesh(core_axis_name="c", subcore_axis_name="s")

    @functools.partial(
        pl.kernel,
        out_type=jax.ShapeDtypeStruct((_B * _M * _C,), jnp.float32),
        mesh=mesh,
        scratch_types=[
            pltpu.VMEM((_PAIRS,), jnp.int32),
            pltpu.VMEM((_C, 128), jnp.float32),
            pltpu.VMEM((_C,), jnp.float32),
            pltpu.SemaphoreType.DMA,
        ],
        compiler_params=pltpu.CompilerParams(needs_layout_passes=False),
    )
    def gather_kernel(t_hbm, i_hbm, o_hbm, idx_v, blk_v, outb, sem):
        wid = jax.lax.axis_index("s") * _SC_NC + jax.lax.axis_index("c")
        pltpu.sync_copy(i_hbm, idx_v)
        lane_iota = jax.lax.iota(jnp.int32, _SC_L)
        for pair_local in range(_PPW):
            p = wid * _PPW + pair_local
            b = p // _M
            idxval = plsc.load_gather(idx_v, [jnp.full((_SC_L,), p, jnp.int32)])
            lane = jnp.bitwise_and(idxval, 127)
            idx_s = jnp.max(idxval)
            col0 = pl.multiple_of((idx_s >> 7) << 7, 128)
            row0 = pl.multiple_of(b * _C, _C)
            pltpu.async_copy(
                t_hbm.at[pl.ds(row0, _C), pl.ds(col0, 128)], blk_v, sem
            ).wait()
            for j in range(_C // _SC_L):
                vals = plsc.load_gather(blk_v, [j * _SC_L + lane_iota, lane])
                outb[pl.ds(j * _SC_L, _SC_L)] = vals
            off = pl.multiple_of(p * _C, _C)
            pltpu.sync_copy(outb, o_hbm.at[pl.ds(off, _C)])

    out = gather_kernel(table, idx_flat)
    return out.reshape(_B, _M, _C).transpose(0, 2, 1)


def kernel(locations, features):
    loc4 = locations.transpose(0, 2, 1).reshape(_B, 3, _ROWS, _LANES)
    anchor_idx, pts = pl.pallas_call(
        _fps_body,
        grid=(_B,),
        in_specs=[
            pl.BlockSpec((1, 3, _ROWS, _LANES), lambda i: (i, 0, 0, 0)),
        ],
        out_specs=(
            pl.BlockSpec((1, 1, _M), lambda i: (i, 0, 0)),
            pl.BlockSpec((1, 1, 3, _M), lambda i: (i, 0, 0, 0)),
        ),
        out_shape=(
            jax.ShapeDtypeStruct((_B, 1, _M), jnp.int32),
            jax.ShapeDtypeStruct((_B, 1, 3, _M), jnp.float32),
        ),
        compiler_params=pltpu.CompilerParams(
            dimension_semantics=("parallel",),
        ),
    )(loc4)
    anchor_idx = anchor_idx.reshape(_B, _M)
    anchor_points = pts.reshape(_B, 3, _M).transpose(0, 2, 1)
    anchor_features = _sc_gather_features(features, anchor_idx)
    return anchor_points, anchor_features, anchor_idx


# speedup vs baseline: 1.5351x; 1.5351x over previous
import functools

import jax
import jax.numpy as jnp
from jax.experimental import pallas as pl
from jax.experimental.pallas import tpu as pltpu
from jax.experimental.pallas import tpu_sc as plsc

_B = 4
_N = 65536
_C = 128
_M = 16
_LANES = 128
_ROWS = _N // _LANES


def _fps_body(loc_ref, idx_ref, pts_ref):
    shape2 = (_ROWS, _LANES)
    row = jax.lax.broadcasted_iota(jnp.int32, shape2, 0)
    col = jax.lax.broadcasted_iota(jnp.int32, shape2, 1)
    lin = row * _LANES + col
    lane3 = jax.lax.broadcasted_iota(jnp.int32, (3, 1, _LANES), 2)
    big = jnp.int32(jnp.iinfo(jnp.int32).max)

    def coords_of(b, n):
        r = n // _LANES
        c = n % _LANES
        rowv = loc_ref[b, :, pl.ds(r, 1), :]
        return jnp.sum(jnp.where(lane3 == c, rowv, 0.0), axis=2, keepdims=True)

    dists = [jnp.full(shape2, 1e10, dtype=jnp.float32) for _ in range(_B)]
    cur = [jnp.zeros((), dtype=jnp.int32) for _ in range(_B)]
    idx_ref[:, 0:1] = jnp.zeros((_B, 1), jnp.int32)

    for i in range(1, _M + 1):
        for b in range(_B):
            l3 = coords_of(b, cur[b])
            pts_ref[b:b + 1, :, i - 1:i] = l3.reshape(1, 3, 1)
            if i == _M:
                continue
            dx = loc_ref[b, 0] - l3[0]
            dy = loc_ref[b, 1] - l3[1]
            dz = loc_ref[b, 2] - l3[2]
            d = (dx * dx + dy * dy) + dz * dz
            dmin = jnp.minimum(dists[b], d)
            dists[b] = dmin
            m = jnp.max(dmin)
            nxt = jnp.min(jnp.where(dmin == m, lin, big)).astype(jnp.int32)
            idx_ref[b:b + 1, i:i + 1] = nxt.reshape(1, 1)
            cur[b] = nxt


_SC_NC = 2
_SC_NS = 16
_SC_L = 16
_GRAN = 16
_NROWS = _N // _GRAN
_PAIRS = _B * _M
_NW = _SC_NC * _SC_NS
_PPW = _PAIRS // _NW


def _sc_gather_features(features, anchor_idx):
    table = features.reshape(_B * _C, _N)
    idx_flat = anchor_idx.reshape(_PAIRS)
    mesh = plsc.VectorSubcoreMesh(core_axis_name="c", subcore_axis_name="s")

    @functools.partial(
        pl.kernel,
        out_type=jax.ShapeDtypeStruct((_B * _M * _C,), jnp.float32),
        mesh=mesh,
        scratch_types=[
            pltpu.VMEM((_PAIRS,), jnp.int32),
            pltpu.VMEM((_C, 128), jnp.float32),
            pltpu.VMEM((_C,), jnp.float32),
            pltpu.SemaphoreType.DMA,
        ],
        compiler_params=pltpu.CompilerParams(needs_layout_passes=False),
    )
    def gather_kernel(t_hbm, i_hbm, o_hbm, idx_v, blk_v, outb, sem):
        wid = jax.lax.axis_index("s") * _SC_NC + jax.lax.axis_index("c")
        pltpu.sync_copy(i_hbm, idx_v)
        lane_iota = jax.lax.iota(jnp.int32, _SC_L)
        for pair_local in range(_PPW):
            p = wid * _PPW + pair_local
            b = p // _M
            idxval = plsc.load_gather(idx_v, [jnp.full((_SC_L,), p, jnp.int32)])
            lane = jnp.bitwise_and(idxval, 127)
            idx_s = jnp.max(idxval)
            col0 = pl.multiple_of((idx_s >> 7) << 7, 128)
            row0 = pl.multiple_of(b * _C, _C)
            pltpu.async_copy(
                t_hbm.at[pl.ds(row0, _C), pl.ds(col0, 128)], blk_v, sem
            ).wait()
            for j in range(_C // _SC_L):
                vals = plsc.load_gather(blk_v, [j * _SC_L + lane_iota, lane])
                outb[pl.ds(j * _SC_L, _SC_L)] = vals
            off = pl.multiple_of(p * _C, _C)
            pltpu.sync_copy(outb, o_hbm.at[pl.ds(off, _C)])

    out = gather_kernel(table, idx_flat)
    return out.reshape(_B, _M, _C).transpose(0, 2, 1)


def kernel(locations, features):
    loc4 = locations.transpose(0, 2, 1).reshape(_B, 3, _ROWS, _LANES)
    anchor_idx, pts = pl.pallas_call(
        _fps_body,
        out_shape=(
            jax.ShapeDtypeStruct((_B, _M), jnp.int32),
            jax.ShapeDtypeStruct((_B, 3, _M), jnp.float32),
        ),
    )(loc4)
    anchor_points = pts.transpose(0, 2, 1)
    anchor_features = _sc_gather_features(features, anchor_idx)
    return anchor_points, anchor_features, anchor_idx


# speedup vs baseline: 1.6374x; 1.0666x over previous
import functools

import jax
import jax.numpy as jnp
from jax.experimental import pallas as pl
from jax.experimental.pallas import tpu as pltpu
from jax.experimental.pallas import tpu_sc as plsc

_B = 4
_N = 65536
_C = 128
_M = 16
_LANES = 128
_ROWS = _N // _LANES


def _fps_body(loc_ref, idx_ref, idx64_ref, pts_ref):
    xs = loc_ref[:, 0]
    ys = loc_ref[:, 1]
    zs = loc_ref[:, 2]
    shape2 = (_ROWS, _LANES)
    row = jax.lax.broadcasted_iota(jnp.int32, shape2, 0)
    col = jax.lax.broadcasted_iota(jnp.int32, shape2, 1)
    lin = row * _LANES + col
    lane3 = jax.lax.broadcasted_iota(jnp.int32, (3, 1, _LANES), 2)
    big = jnp.int32(jnp.iinfo(jnp.int32).max)

    def coords_of(b, n):
        r = n // _LANES
        c = n % _LANES
        rowv = loc_ref[b, :, pl.ds(r, 1), :]
        return jnp.sum(jnp.where(lane3 == c, rowv, 0.0), axis=2, keepdims=True)

    dists = jnp.full((_B,) + shape2, 1e10, dtype=jnp.float32)
    cur = [jnp.zeros((), dtype=jnp.int32) for _ in range(_B)]
    idx_ref[:, 0:1] = jnp.zeros((_B, 1), jnp.int32)
    idx64_ref[...] = jnp.zeros((1, _B * _M), jnp.int32)

    for i in range(1, _M + 1):
        l3s = [coords_of(b, cur[b]) for b in range(_B)]
        for b in range(_B):
            pts_ref[b:b + 1, i - 1:i, :] = l3s[b].reshape(1, 1, 3)
        if i == _M:
            break
        lx = jnp.concatenate([l3[0:1] for l3 in l3s], axis=0)
        ly = jnp.concatenate([l3[1:2] for l3 in l3s], axis=0)
        lz = jnp.concatenate([l3[2:3] for l3 in l3s], axis=0)
        dx = xs - lx
        dy = ys - ly
        dz = zs - lz
        d = (dx * dx + dy * dy) + dz * dz
        dmin = jnp.minimum(dists, d)
        dists = dmin
        m = jnp.max(dmin, axis=(1, 2), keepdims=True)
        nxtv = jnp.min(jnp.where(dmin == m, lin, big), axis=(1, 2))
        nxtv = nxtv.astype(jnp.int32)
        idx_ref[:, i:i + 1] = nxtv.reshape(_B, 1)
        for b in range(_B):
            idx64_ref[0:1, b * _M + i:b * _M + i + 1] = nxtv[b:b + 1].reshape(1, 1)
            cur[b] = jnp.max(nxtv[b:b + 1])


_SC_NC = 2
_SC_NS = 16
_SC_L = 16
_PAIRS = _B * _M
_NW = _SC_NC * _SC_NS
_PPW = _PAIRS // _NW


def _sc_gather_features(features, idx64):
    table = features.reshape(_B * _C, _N)
    mesh = plsc.VectorSubcoreMesh(core_axis_name="c", subcore_axis_name="s")

    @functools.partial(
        pl.kernel,
        out_type=jax.ShapeDtypeStruct((_B * _M * _C,), jnp.float32),
        mesh=mesh,
        scratch_types=[
            pltpu.VMEM((_PAIRS,), jnp.int32),
            pltpu.VMEM((_C, 128), jnp.float32),
            pltpu.VMEM((_C,), jnp.float32),
            pltpu.SemaphoreType.DMA,
        ],
        compiler_params=pltpu.CompilerParams(needs_layout_passes=False),
    )
    def gather_kernel(t_hbm, i_hbm, o_hbm, idx_v, blk_v, outb, sem):
        wid = jax.lax.axis_index("s") * _SC_NC + jax.lax.axis_index("c")
        pltpu.sync_copy(i_hbm.at[0], idx_v)
        lane_iota = jax.lax.iota(jnp.int32, _SC_L)
        for pair_local in range(_PPW):
            p = wid * _PPW + pair_local
            b = p // _M
            idxval = plsc.load_gather(idx_v, [jnp.full((_SC_L,), p, jnp.int32)])
            lane = jnp.bitwise_and(idxval, 127)
            idx_s = jnp.max(idxval)
            col0 = pl.multiple_of((idx_s >> 7) << 7, 128)
            row0 = pl.multiple_of(b * _C, _C)
            pltpu.async_copy(
                t_hbm.at[pl.ds(row0, _C), pl.ds(col0, 128)], blk_v, sem
            ).wait()
            for j in range(_C // _SC_L):
                vals = plsc.load_gather(blk_v, [j * _SC_L + lane_iota, lane])
                outb[pl.ds(j * _SC_L, _SC_L)] = vals
            off = pl.multiple_of(p * _C, _C)
            pltpu.sync_copy(outb, o_hbm.at[pl.ds(off, _C)])

    out = gather_kernel(table, idx64)
    return out.reshape(_B, _M, _C).transpose(0, 2, 1)


def kernel(locations, features):
    loc4 = locations.transpose(0, 2, 1).reshape(_B, 3, _ROWS, _LANES)
    anchor_idx, idx64, anchor_points = pl.pallas_call(
        _fps_body,
        out_shape=(
            jax.ShapeDtypeStruct((_B, _M), jnp.int32),
            jax.ShapeDtypeStruct((1, _B * _M), jnp.int32),
            jax.ShapeDtypeStruct((_B, _M, 3), jnp.float32),
        ),
    )(loc4)
    anchor_features = _sc_gather_features(features, idx64)
    return anchor_points, anchor_features, anchor_idx


# speedup vs baseline: 1.6673x; 1.0183x over previous
import functools

import jax
import jax.numpy as jnp
from jax.experimental import pallas as pl
from jax.experimental.pallas import tpu as pltpu
from jax.experimental.pallas import tpu_sc as plsc

_B = 4
_N = 65536
_C = 128
_M = 16
_LANES = 128
_ROWS = _N // _LANES


def _fps_body(loc_ref, idx64_ref, pts_ref):
    xs = loc_ref[:, 0]
    ys = loc_ref[:, 1]
    zs = loc_ref[:, 2]
    shape2 = (_ROWS, _LANES)
    row = jax.lax.broadcasted_iota(jnp.int32, shape2, 0)
    col = jax.lax.broadcasted_iota(jnp.int32, shape2, 1)
    lin = row * _LANES + col
    lane3 = jax.lax.broadcasted_iota(jnp.int32, (3, 1, _LANES), 2)
    big = jnp.int32(jnp.iinfo(jnp.int32).max)

    def coords_of(b, n):
        r = n // _LANES
        c = n % _LANES
        rowv = loc_ref[b, :, pl.ds(r, 1), :]
        return jnp.sum(jnp.where(lane3 == c, rowv, 0.0), axis=2, keepdims=True)

    dists = jnp.full((_B,) + shape2, 1e10, dtype=jnp.float32)
    cur = [jnp.zeros((), dtype=jnp.int32) for _ in range(_B)]
    idx64_ref[...] = jnp.zeros((1, _B * _M), jnp.int32)

    for i in range(1, _M + 1):
        l3s = [coords_of(b, cur[b]) for b in range(_B)]
        for b in range(_B):
            pts_ref[b:b + 1, i - 1:i, :] = l3s[b].reshape(1, 1, 3)
        if i == _M:
            break
        lx = jnp.concatenate([l3[0:1] for l3 in l3s], axis=0)
        ly = jnp.concatenate([l3[1:2] for l3 in l3s], axis=0)
        lz = jnp.concatenate([l3[2:3] for l3 in l3s], axis=0)
        dx = xs - lx
        dy = ys - ly
        dz = zs - lz
        d = (dx * dx + dy * dy) + dz * dz
        dmin = jnp.minimum(dists, d)
        dists = dmin
        m = jnp.max(dmin, axis=(1, 2), keepdims=True)
        nxtv = jnp.min(jnp.where(dmin == m, lin, big), axis=(1, 2))
        nxtv = nxtv.astype(jnp.int32)
        for b in range(_B):
            idx64_ref[0:1, b * _M + i:b * _M + i + 1] = nxtv[b:b + 1].reshape(1, 1)
            cur[b] = jnp.max(nxtv[b:b + 1])


_SC_NC = 2
_SC_NS = 16
_SC_L = 16
_PAIRS = _B * _M
_NW = _SC_NC * _SC_NS
_PPW = _PAIRS // _NW


def _sc_gather_features(features, idx64):
    table = features.reshape(_B * _C, _N)
    mesh = plsc.VectorSubcoreMesh(core_axis_name="c", subcore_axis_name="s")

    @functools.partial(
        pl.kernel,
        out_type=jax.ShapeDtypeStruct((_B * _M * _C,), jnp.float32),
        mesh=mesh,
        scratch_types=[
            pltpu.VMEM((_PAIRS,), jnp.int32),
            pltpu.VMEM((_C, 128), jnp.float32),
            pltpu.VMEM((_C, 128), jnp.float32),
            pltpu.VMEM((_C,), jnp.float32),
            pltpu.VMEM((_C,), jnp.float32),
            pltpu.SemaphoreType.DMA,
            pltpu.SemaphoreType.DMA,
        ],
        compiler_params=pltpu.CompilerParams(needs_layout_passes=False),
    )
    def gather_kernel(t_hbm, i_hbm, o_hbm, idx_v, blk0, blk1, ob0, ob1,
                      sem, osem):
        wid = jax.lax.axis_index("s") * _SC_NC + jax.lax.axis_index("c")
        pltpu.sync_copy(i_hbm.at[0], idx_v)
        lane_iota = jax.lax.iota(jnp.int32, _SC_L)
        blks = (blk0, blk1)
        obs = (ob0, ob1)
        lanes = []
        copies = []
        for pair_local in range(_PPW):
            p = wid * _PPW + pair_local
            b = p // _M
            idxval = plsc.load_gather(idx_v, [jnp.full((_SC_L,), p, jnp.int32)])
            lanes.append(jnp.bitwise_and(idxval, 127))
            idx_s = jnp.max(idxval)
            col0 = pl.multiple_of((idx_s >> 7) << 7, 128)
            row0 = pl.multiple_of(b * _C, _C)
            copies.append(pltpu.async_copy(
                t_hbm.at[pl.ds(row0, _C), pl.ds(col0, 128)], blks[pair_local],
                sem,
            ))
        outcopies = []
        for pair_local in range(_PPW):
            p = wid * _PPW + pair_local
            copies[pair_local].wait()
            for j in range(_C // _SC_L):
                vals = plsc.load_gather(
                    blks[pair_local], [j * _SC_L + lane_iota, lanes[pair_local]]
                )
                obs[pair_local][pl.ds(j * _SC_L, _SC_L)] = vals
            off = pl.multiple_of(p * _C, _C)
            outcopies.append(pltpu.async_copy(
                obs[pair_local], o_hbm.at[pl.ds(off, _C)], osem,
            ))
        for cp in outcopies:
            cp.wait()

    out = gather_kernel(table, idx64)
    return out.reshape(_B, _M, _C).transpose(0, 2, 1)


def kernel(locations, features):
    loc4 = locations.transpose(0, 2, 1).reshape(_B, 3, _ROWS, _LANES)
    idx64, anchor_points = pl.pallas_call(
        _fps_body,
        out_shape=(
            jax.ShapeDtypeStruct((1, _B * _M), jnp.int32),
            jax.ShapeDtypeStruct((_B, _M, 3), jnp.float32),
        ),
    )(loc4)
    anchor_features = _sc_gather_features(features, idx64)
    anchor_idx = idx64.reshape(_B, _M)
    return anchor_points, anchor_features, anchor_idx
